# baseline (speedup 1.0000x reference)
import functools

import jax
import jax.numpy as jnp
from jax.experimental import pallas as pl
from jax.experimental.pallas import tpu as pltpu

_N = 8192
_D = 16
_P = 256
_N_BLK = 2048
_NB = _N // _N_BLK
_Q_MIN = 0.01
_REP = 10.0


def _fused_kernel(beta_ref, pid_ref, x_ref, out_ref,
                  xat, bestq, racc, tacc):
    ph = pl.program_id(0)
    b = pl.program_id(1)

    beta_col = beta_ref[...]
    at = 0.5 * jnp.log((1.0 + beta_col) / (1.0 - beta_col))
    q_col = at * at + _Q_MIN

    lane = jax.lax.broadcasted_iota(jnp.int32, (_N_BLK, _P), 1)
    mask = pid_ref[...] == (lane + 1)
    x_blk = x_ref[...]

    @pl.when(jnp.logical_and(ph == 0, b == 0))
    def _init0():
        xat[...] = jnp.zeros((_D, _P), jnp.float32)
        bestq[...] = jnp.full((1, _P), -1.0, jnp.float32)
        racc[...] = jnp.zeros((1, _P), jnp.float32)
        tacc[...] = jnp.zeros((1, _P), jnp.float32)

    @pl.when(ph == 0)
    def _select():
        n_loc = jax.lax.broadcasted_iota(jnp.int32, (_N_BLK, _P), 0)
        mq = jnp.where(mask, q_col, -1.0)
        bmax = jnp.max(mq, axis=0, keepdims=True)
        nidx = jnp.where(mq == bmax, n_loc, _N)
        bmin = jnp.min(nidx, axis=0, keepdims=True)
        upd = bmax > bestq[...]
        sel = jnp.logical_and(n_loc == bmin, upd).astype(jnp.float32)
        xcand = jax.lax.dot_general(
            x_blk, sel, (((0,), (0,)), ((), ())),
            preferred_element_type=jnp.float32)
        xat[...] = jnp.where(upd, xcand, xat[...])
        bestq[...] = jnp.where(upd, bmax, bestq[...])

    @pl.when(ph == 1)
    def _dense():
        xa = xat[...]
        dot = jax.lax.dot_general(
            x_blk, xa, (((1,), (0,)), ((), ())),
            preferred_element_type=jnp.float32)
        xn2 = jnp.sum(x_blk * x_blk, axis=1, keepdims=True)
        xa2 = jnp.sum(xa * xa, axis=0, keepdims=True)
        d2 = jnp.maximum(xn2 + xa2 - 2.0 * dot, 0.0)
        hinge = jnp.maximum(1.0 - jnp.sqrt(d2), 0.0)
        seg = jnp.where(mask, d2 - _REP * hinge, 0.0)
        racc[...] += jax.lax.dot_general(
            q_col, hinge, (((0,), (0,)), ((), ())),
            preferred_element_type=jnp.float32)
        tacc[...] += jax.lax.dot_general(
            q_col, seg, (((0,), (0,)), ((), ())),
            preferred_element_type=jnp.float32)

        @pl.when(b == _NB - 1)
        def _final():
            bq = bestq[...]
            present = (bq >= 0.0).astype(jnp.float32)
            s = bq * (tacc[...] + _REP * racc[...]) * present
            out_ref[...] = jnp.sum(s, axis=(0, 1), keepdims=True) / _N


@functools.partial(jax.jit)
def _potential_loss(beta, x, particle_id):
    beta2 = beta.reshape(_N, 1)
    pid2 = particle_id.reshape(_N, 1)
    out = pl.pallas_call(
        _fused_kernel,
        grid=(2, _NB),
        in_specs=[
            pl.BlockSpec((_N_BLK, 1), lambda ph, b: (b, 0)),
            pl.BlockSpec((_N_BLK, 1), lambda ph, b: (b, 0)),
            pl.BlockSpec((_N_BLK, _D), lambda ph, b: (b, 0)),
        ],
        out_specs=pl.BlockSpec((1, 1), lambda ph, b: (0, 0)),
        out_shape=jax.ShapeDtypeStruct((1, 1), jnp.float32),
        scratch_shapes=[
            pltpu.VMEM((_D, _P), jnp.float32),
            pltpu.VMEM((1, _P), jnp.float32),
            pltpu.VMEM((1, _P), jnp.float32),
            pltpu.VMEM((1, _P), jnp.float32),
        ],
        compiler_params=pltpu.CompilerParams(
            dimension_semantics=("arbitrary", "arbitrary"),
        ),
    )(beta2, pid2, x)
    return out[0, 0]


def kernel(w, beta, x, y, particle_id):
    return _potential_loss(beta, x, particle_id)
